# baseline (speedup 1.0000x reference)
import jax
import jax.numpy as jnp
from jax import lax
from jax.experimental import pallas as pl
from jax.experimental.pallas import tpu as pltpu, tpu_sc as plsc

Q_TOTAL = 65536
N_SRC = 100000
K_NB = 64
SIGMA_N_CONST = 0.8
EPS_CONST = 1e-08
NUM_TILES = 32
QPT = Q_TOTAL // NUM_TILES
BLK_Q = 16
NBLK = QPT // BLK_Q
ROWS_PER_BLK = BLK_Q * K_NB
IDX_CHUNK = 128
NCHUNK = ROWS_PER_BLK // IDX_CHUNK
UNROLL = 16

_NEG_INV_SIG = -1.0 / (SIGMA_N_CONST * SIGMA_N_CONST + EPS_CONST)


def _sc_body(packed_hbm, qx_hbm, qy_hbm, qz_hbm, h_hbm, idx_hbm,
             pot_hbm, gx_hbm, gy_hbm, gz_hbm,
             idx_v, rows_v, der_v, q_v, out_v, gsem, isem):
    nc = 2
    wid = lax.axis_index("s") * nc + lax.axis_index("c")
    qbase = pl.multiple_of(wid * QPT, QPT)

    for i, src_h in enumerate((qx_hbm, qy_hbm, qz_hbm, h_hbm)):
        pltpu.sync_copy(src_h.at[pl.ds(qbase, QPT)], q_v.at[i])

    lane = lax.broadcasted_iota(jnp.int32, (16,), 0)
    lane64 = lane * K_NB
    cols = [jnp.full((16,), c, jnp.int32) for c in range(6)]
    neg_inv_sig = jnp.full((16,), _NEG_INV_SIG, jnp.float32)
    nmax = jnp.full((16,), N_SRC - 1, jnp.int32)
    zero = jnp.zeros((16,), jnp.int32)

    def idx_slice(b):
        g0 = qbase + b * BLK_Q
        irow = pl.multiple_of(g0 // 2, NCHUNK)
        return idx_hbm.at[pl.ds(irow, NCHUNK)]

    def fire_idx(b, par):
        pltpu.async_copy(idx_slice(b), idx_v.at[par], isem)

    def wait_idx(b, par):
        pltpu.make_async_copy(idx_slice(b), idx_v.at[par], isem).wait()

    def fire_rows(b, par):
        for r in range(NCHUNK):
            for c in range(IDX_CHUNK // 16):
                sl = (par, r, pl.ds(c * 16, 16))
                v = idx_v[sl]
                idx_v[sl] = jnp.minimum(jnp.maximum(v, zero), nmax)
        for j in range(NCHUNK):
            pltpu.async_copy(
                packed_hbm.at[idx_v.at[par, j]],
                rows_v.at[par, pl.ds(j * IDX_CHUNK, IDX_CHUNK)],
                gsem,
            )

    def wait_rows(par):
        for j in range(NCHUNK):
            pltpu.make_async_copy(
                packed_hbm.at[idx_v.at[par, j]],
                rows_v.at[par, pl.ds(j * IDX_CHUNK, IDX_CHUNK)],
                gsem,
            ).wait()

    fire_idx(0, 0)
    wait_idx(0, 0)
    fire_rows(0, 0)
    fire_idx(1, 1)

    def block_body(b, _):
        par = lax.rem(b, 2)
        nxt = 1 - par
        rows_b = rows_v.at[par]

        @pl.when(b + 1 < NBLK)
        def _prefetch_rows():
            wait_idx(b + 1, nxt)
            fire_rows(b + 1, nxt)

        wait_rows(par)

        @pl.when(b + 2 < NBLK)
        def _prefetch_idx():
            fire_idx(b + 2, par)

        qsl = pl.ds(b * BLK_Q, BLK_Q)
        qx = q_v[0, qsl]
        qy = q_v[1, qsl]
        qz = q_v[2, qsl]
        h = q_v[3, qsl]
        neg_inv_h2 = -1.0 / (h * h + EPS_CONST)
        two_inv = 2.0 * neg_inv_h2

        zf = jnp.zeros((16,), jnp.float32)
        acc0 = (zf,) * 11

        def base_k(k, acc):
            (sw, swf, sgx, sgy, sgz, sfx, sfy, sfz, swx, swy, swz) = acc
            row = lane64 + k
            nbx = plsc.load_gather(rows_b, [row, cols[0]])
            nby = plsc.load_gather(rows_b, [row, cols[1]])
            nbz = plsc.load_gather(rows_b, [row, cols[2]])
            nnx = plsc.load_gather(rows_b, [row, cols[3]])
            nny = plsc.load_gather(rows_b, [row, cols[4]])
            nnz = plsc.load_gather(rows_b, [row, cols[5]])
            dx = qx - nbx
            dy = qy - nby
            dz = qz - nbz
            d2 = dx * dx + dy * dy + dz * dz
            bw = jnp.exp(d2 * neg_inv_h2)
            f = dx * nnx + dy * nny + dz * nnz
            s = bw * two_inv
            bgx = s * dx
            bgy = s * dy
            bgz = s * dz
            der_v[0, k, :] = bw
            der_v[1, k, :] = f
            der_v[2, k, :] = bgx
            der_v[3, k, :] = bgy
            der_v[4, k, :] = bgz
            der_v[5, k, :] = nnx
            der_v[6, k, :] = nny
            der_v[7, k, :] = nnz
            return (sw + bw, swf + bw * f,
                    sgx + bgx, sgy + bgy, sgz + bgz,
                    sfx + bgx * f, sfy + bgy * f, sfz + bgz * f,
                    swx + bw * nnx, swy + bw * nny, swz + bw * nnz)

        acc = plsc.parallel_loop(0, K_NB, 1, unroll=UNROLL, carry=acc0)(base_k)

        def finish(acc):
            (sw, swf, sgx, sgy, sgz, sfx, sfy, sfz, swx, swy, swz) = acc
            swe = sw + EPS_CONST
            inv = 1.0 / swe
            pot = swf * inv
            gx = (sfx + swx - sgx * pot) * inv
            gy = (sfy + swy - sgy * pot) * inv
            gz = (sfz + swz - sgz * pot) * inv
            return pot, gx, gy, gz

        pot, gx, gy, gz = finish(acc)

        for _ in range(2):
            def iter_k(k, acc, gx=gx, gy=gy, gz=gz):
                (sw, swf, sgx, sgy, sgz,
                 sfx, sfy, sfz, swx, swy, swz) = acc
                bw = der_v[0, k, :]
                f = der_v[1, k, :]
                bgx = der_v[2, k, :]
                bgy = der_v[3, k, :]
                bgz = der_v[4, k, :]
                nnx = der_v[5, k, :]
                nny = der_v[6, k, :]
                nnz = der_v[7, k, :]
                tx = nnx - gx
                ty = nny - gy
                tz = nnz - gz
                nd2 = tx * tx + ty * ty + tz * tz
                rw = jnp.exp(nd2 * neg_inv_sig)
                w = bw * rw
                wgx = bgx * rw
                wgy = bgy * rw
                wgz = bgz * rw
                return (sw + w, swf + w * f,
                        sgx + wgx, sgy + wgy, sgz + wgz,
                        sfx + wgx * f, sfy + wgy * f, sfz + wgz * f,
                        swx + w * nnx, swy + w * nny, swz + w * nnz)

            acc = plsc.parallel_loop(0, K_NB, 1, unroll=UNROLL, carry=acc0)(iter_k)
            pot, gx, gy, gz = finish(acc)

        out_v[0, qsl] = pot
        out_v[1, qsl] = gx
        out_v[2, qsl] = gy
        out_v[3, qsl] = gz
        return _

    lax.fori_loop(0, NBLK, block_body, 0)

    for i, dst_h in enumerate((pot_hbm, gx_hbm, gy_hbm, gz_hbm)):
        pltpu.sync_copy(out_v.at[i], dst_h.at[pl.ds(qbase, QPT)])


@jax.jit
def _run(packed, qx, qy, qz, h, idx2):
    mesh = plsc.VectorSubcoreMesh(core_axis_name="c", subcore_axis_name="s")
    kfn = pl.kernel(
        _sc_body,
        out_type=tuple(
            jax.ShapeDtypeStruct((Q_TOTAL,), jnp.float32) for _ in range(4)
        ),
        mesh=mesh,
        compiler_params=pltpu.CompilerParams(
            needs_layout_passes=False, use_tc_tiling_on_sc=False),
        scratch_types=(
            pltpu.VMEM((2, NCHUNK, IDX_CHUNK), jnp.int32),
            pltpu.VMEM((2, ROWS_PER_BLK, 8), jnp.float32),
            pltpu.VMEM((8, K_NB, 16), jnp.float32),
            pltpu.VMEM((4, QPT), jnp.float32),
            pltpu.VMEM((4, QPT), jnp.float32),
            pltpu.SemaphoreType.DMA,
            pltpu.SemaphoreType.DMA,
        ),
    )
    return kfn(packed, qx, qy, qz, h, idx2)


def kernel(query_points, source_vertices, source_normals, neighbor_indices,
           bandwidth_h, compute_gradient):
    n = source_vertices.shape[0]
    packed = jnp.concatenate(
        [source_vertices, source_normals,
         jnp.zeros((n, 2), jnp.float32)], axis=1)
    idx2 = neighbor_indices.astype(jnp.int32).reshape(-1, IDX_CHUNK)
    pot, gx, gy, gz = _run(
        packed, query_points[:, 0], query_points[:, 1], query_points[:, 2],
        bandwidth_h, idx2)
    grad = jnp.stack([gx, gy, gz], axis=1)
    grad = jnp.where(compute_gradient != 0, grad, jnp.zeros_like(grad))
    return (pot, grad)


# speedup vs baseline: 1.3980x; 1.3980x over previous
import jax
import jax.numpy as jnp
from jax import lax
from jax.experimental import pallas as pl
from jax.experimental.pallas import tpu as pltpu, tpu_sc as plsc

Q_TOTAL = 65536
N_SRC = 100000
K_NB = 64
SIGMA_N_CONST = 0.8
EPS_CONST = 1e-08
NUM_TILES = 32
QPT = Q_TOTAL // NUM_TILES
BLK_Q = 16
NBLK = QPT // BLK_Q
ROWS_PER_BLK = BLK_Q * K_NB
IDX_CHUNK = 128
NCHUNK = ROWS_PER_BLK // IDX_CHUNK
UNROLL = 4

_NEG_INV_SIG = -1.0 / (SIGMA_N_CONST * SIGMA_N_CONST + EPS_CONST)


def _sc_body(packed_hbm, qx_hbm, qy_hbm, qz_hbm, h_hbm, idx_hbm,
             pot_hbm, gx_hbm, gy_hbm, gz_hbm,
             idx_v, rows_v, der_v, q_v, out_v, gsem, isem):
    nc = 2
    wid = lax.axis_index("s") * nc + lax.axis_index("c")
    qbase = pl.multiple_of(wid * QPT, QPT)

    for i, src_h in enumerate((qx_hbm, qy_hbm, qz_hbm, h_hbm)):
        pltpu.sync_copy(src_h.at[pl.ds(qbase, QPT)], q_v.at[i])

    lane = lax.broadcasted_iota(jnp.int32, (16,), 0)
    lane64 = lane * K_NB
    cols = [jnp.full((16,), c, jnp.int32) for c in range(6)]
    neg_inv_sig = jnp.full((16,), _NEG_INV_SIG, jnp.float32)
    nmax = jnp.full((16,), N_SRC - 1, jnp.int32)
    zero = jnp.zeros((16,), jnp.int32)

    def idx_slice(b):
        g0 = qbase + b * BLK_Q
        irow = pl.multiple_of(g0 // 2, NCHUNK)
        return idx_hbm.at[pl.ds(irow, NCHUNK)]

    def fire_idx(b, par):
        pltpu.async_copy(idx_slice(b), idx_v.at[par], isem)

    def wait_idx(b, par):
        pltpu.make_async_copy(idx_slice(b), idx_v.at[par], isem).wait()

    def fire_rows(b, par):
        for r in range(NCHUNK):
            for c in range(IDX_CHUNK // 16):
                sl = (par, r, pl.ds(c * 16, 16))
                v = idx_v[sl]
                idx_v[sl] = jnp.minimum(jnp.maximum(v, zero), nmax)
        for j in range(NCHUNK):
            pltpu.async_copy(
                packed_hbm.at[idx_v.at[par, j]],
                rows_v.at[par, pl.ds(j * IDX_CHUNK, IDX_CHUNK)],
                gsem,
            )

    def wait_rows(par):
        for j in range(NCHUNK):
            pltpu.make_async_copy(
                packed_hbm.at[idx_v.at[par, j]],
                rows_v.at[par, pl.ds(j * IDX_CHUNK, IDX_CHUNK)],
                gsem,
            ).wait()

    fire_idx(0, 0)
    wait_idx(0, 0)
    fire_rows(0, 0)
    fire_idx(1, 1)

    def block_body(b, _):
        par = lax.rem(b, 2)
        nxt = 1 - par
        rows_b = rows_v.at[par]

        @pl.when(b + 1 < NBLK)
        def _prefetch_rows():
            wait_idx(b + 1, nxt)
            fire_rows(b + 1, nxt)

        wait_rows(par)

        @pl.when(b + 2 < NBLK)
        def _prefetch_idx():
            fire_idx(b + 2, par)

        qsl = pl.ds(b * BLK_Q, BLK_Q)
        qx = q_v[0, qsl]
        qy = q_v[1, qsl]
        qz = q_v[2, qsl]
        h = q_v[3, qsl]
        neg_inv_h2 = -1.0 / (h * h + EPS_CONST)
        two_inv = 2.0 * neg_inv_h2

        zf = jnp.zeros((16,), jnp.float32)
        acc0 = (zf,) * 11

        def base_k(k, acc):
            (sw, swf, sgx, sgy, sgz, sfx, sfy, sfz, swx, swy, swz) = acc
            row = lane64 + k
            nbx = plsc.load_gather(rows_b, [row, cols[0]])
            nby = plsc.load_gather(rows_b, [row, cols[1]])
            nbz = plsc.load_gather(rows_b, [row, cols[2]])
            nnx = plsc.load_gather(rows_b, [row, cols[3]])
            nny = plsc.load_gather(rows_b, [row, cols[4]])
            nnz = plsc.load_gather(rows_b, [row, cols[5]])
            dx = qx - nbx
            dy = qy - nby
            dz = qz - nbz
            d2 = dx * dx + dy * dy + dz * dz
            bw = jnp.exp(d2 * neg_inv_h2)
            f = dx * nnx + dy * nny + dz * nnz
            s = bw * two_inv
            bgx = s * dx
            bgy = s * dy
            bgz = s * dz
            der_v[0, k, :] = bw
            der_v[1, k, :] = f
            der_v[2, k, :] = bgx
            der_v[3, k, :] = bgy
            der_v[4, k, :] = bgz
            der_v[5, k, :] = nnx
            der_v[6, k, :] = nny
            der_v[7, k, :] = nnz
            return (sw + bw, swf + bw * f,
                    sgx + bgx, sgy + bgy, sgz + bgz,
                    sfx + bgx * f, sfy + bgy * f, sfz + bgz * f,
                    swx + bw * nnx, swy + bw * nny, swz + bw * nnz)

        acc = plsc.parallel_loop(0, K_NB, 1, unroll=UNROLL, carry=acc0)(base_k)

        def finish(acc):
            (sw, swf, sgx, sgy, sgz, sfx, sfy, sfz, swx, swy, swz) = acc
            swe = sw + EPS_CONST
            inv = 1.0 / swe
            pot = swf * inv
            gx = (sfx + swx - sgx * pot) * inv
            gy = (sfy + swy - sgy * pot) * inv
            gz = (sfz + swz - sgz * pot) * inv
            return pot, gx, gy, gz

        pot, gx, gy, gz = finish(acc)

        for _ in range(2):
            def iter_k(k, acc, gx=gx, gy=gy, gz=gz):
                (sw, swf, sgx, sgy, sgz,
                 sfx, sfy, sfz, swx, swy, swz) = acc
                bw = der_v[0, k, :]
                f = der_v[1, k, :]
                bgx = der_v[2, k, :]
                bgy = der_v[3, k, :]
                bgz = der_v[4, k, :]
                nnx = der_v[5, k, :]
                nny = der_v[6, k, :]
                nnz = der_v[7, k, :]
                tx = nnx - gx
                ty = nny - gy
                tz = nnz - gz
                nd2 = tx * tx + ty * ty + tz * tz
                rw = jnp.exp(nd2 * neg_inv_sig)
                w = bw * rw
                wgx = bgx * rw
                wgy = bgy * rw
                wgz = bgz * rw
                return (sw + w, swf + w * f,
                        sgx + wgx, sgy + wgy, sgz + wgz,
                        sfx + wgx * f, sfy + wgy * f, sfz + wgz * f,
                        swx + w * nnx, swy + w * nny, swz + w * nnz)

            acc = plsc.parallel_loop(0, K_NB, 1, unroll=UNROLL, carry=acc0)(iter_k)
            pot, gx, gy, gz = finish(acc)

        out_v[0, qsl] = pot
        out_v[1, qsl] = gx
        out_v[2, qsl] = gy
        out_v[3, qsl] = gz
        return _

    lax.fori_loop(0, NBLK, block_body, 0)

    for i, dst_h in enumerate((pot_hbm, gx_hbm, gy_hbm, gz_hbm)):
        pltpu.sync_copy(out_v.at[i], dst_h.at[pl.ds(qbase, QPT)])


@jax.jit
def _run(packed, qx, qy, qz, h, idx2):
    mesh = plsc.VectorSubcoreMesh(core_axis_name="c", subcore_axis_name="s")
    kfn = pl.kernel(
        _sc_body,
        out_type=tuple(
            jax.ShapeDtypeStruct((Q_TOTAL,), jnp.float32) for _ in range(4)
        ),
        mesh=mesh,
        compiler_params=pltpu.CompilerParams(
            needs_layout_passes=False, use_tc_tiling_on_sc=False),
        scratch_types=(
            pltpu.VMEM((2, NCHUNK, IDX_CHUNK), jnp.int32),
            pltpu.VMEM((2, ROWS_PER_BLK, 8), jnp.float32),
            pltpu.VMEM((8, K_NB, 16), jnp.float32),
            pltpu.VMEM((4, QPT), jnp.float32),
            pltpu.VMEM((4, QPT), jnp.float32),
            pltpu.SemaphoreType.DMA,
            pltpu.SemaphoreType.DMA,
        ),
    )
    return kfn(packed, qx, qy, qz, h, idx2)


def kernel(query_points, source_vertices, source_normals, neighbor_indices,
           bandwidth_h, compute_gradient):
    n = source_vertices.shape[0]
    packed = jnp.concatenate(
        [source_vertices, source_normals,
         jnp.zeros((n, 2), jnp.float32)], axis=1)
    idx2 = neighbor_indices.astype(jnp.int32).reshape(-1, IDX_CHUNK)
    pot, gx, gy, gz = _run(
        packed, query_points[:, 0], query_points[:, 1], query_points[:, 2],
        bandwidth_h, idx2)
    grad = jnp.stack([gx, gy, gz], axis=1)
    grad = jnp.where(compute_gradient != 0, grad, jnp.zeros_like(grad))
    return (pot, grad)


# speedup vs baseline: 1.4741x; 1.0544x over previous
import jax
import jax.numpy as jnp
from jax import lax
from jax.experimental import pallas as pl
from jax.experimental.pallas import tpu as pltpu, tpu_sc as plsc

Q_TOTAL = 65536
N_SRC = 100000
K_NB = 64
SIGMA_N_CONST = 0.8
EPS_CONST = 1e-08
NUM_TILES = 32
QPT = Q_TOTAL // NUM_TILES
BLK_Q = 16
NBLK = QPT // BLK_Q
ROWS_PER_BLK = BLK_Q * K_NB
IDX_CHUNK = 128
NCHUNK = ROWS_PER_BLK // IDX_CHUNK
UNROLL = 2

_NEG_INV_SIG = -1.0 / (SIGMA_N_CONST * SIGMA_N_CONST + EPS_CONST)


def _sc_body(packed_hbm, qx_hbm, qy_hbm, qz_hbm, h_hbm, idx_hbm,
             pot_hbm, gx_hbm, gy_hbm, gz_hbm,
             idx_v, rows_v, der_v, q_v, out_v, gsem, isem):
    nc = 2
    wid = lax.axis_index("s") * nc + lax.axis_index("c")
    qbase = pl.multiple_of(wid * QPT, QPT)

    for i, src_h in enumerate((qx_hbm, qy_hbm, qz_hbm, h_hbm)):
        pltpu.sync_copy(src_h.at[pl.ds(qbase, QPT)], q_v.at[i])

    lane = lax.broadcasted_iota(jnp.int32, (16,), 0)
    lane64 = lane * K_NB
    cols = [jnp.full((16,), c, jnp.int32) for c in range(6)]
    neg_inv_sig = jnp.full((16,), _NEG_INV_SIG, jnp.float32)
    nmax = jnp.full((16,), N_SRC - 1, jnp.int32)
    zero = jnp.zeros((16,), jnp.int32)

    def idx_slice(b):
        g0 = qbase + b * BLK_Q
        irow = pl.multiple_of(g0 // 2, NCHUNK)
        return idx_hbm.at[pl.ds(irow, NCHUNK)]

    def fire_idx(b, par):
        pltpu.async_copy(idx_slice(b), idx_v.at[par], isem)

    def wait_idx(b, par):
        pltpu.make_async_copy(idx_slice(b), idx_v.at[par], isem).wait()

    def fire_rows(b, par):
        for r in range(NCHUNK):
            for c in range(IDX_CHUNK // 16):
                sl = (par, r, pl.ds(c * 16, 16))
                v = idx_v[sl]
                idx_v[sl] = jnp.minimum(jnp.maximum(v, zero), nmax)
        for j in range(NCHUNK):
            pltpu.async_copy(
                packed_hbm.at[idx_v.at[par, j]],
                rows_v.at[par, pl.ds(j * IDX_CHUNK, IDX_CHUNK)],
                gsem,
            )

    def wait_rows(par):
        for j in range(NCHUNK):
            pltpu.make_async_copy(
                packed_hbm.at[idx_v.at[par, j]],
                rows_v.at[par, pl.ds(j * IDX_CHUNK, IDX_CHUNK)],
                gsem,
            ).wait()

    fire_idx(0, 0)
    wait_idx(0, 0)
    fire_rows(0, 0)
    fire_idx(1, 1)

    def block_body(b, _):
        par = lax.rem(b, 2)
        nxt = 1 - par
        rows_b = rows_v.at[par]

        @pl.when(b + 1 < NBLK)
        def _prefetch_rows():
            wait_idx(b + 1, nxt)
            fire_rows(b + 1, nxt)

        wait_rows(par)

        @pl.when(b + 2 < NBLK)
        def _prefetch_idx():
            fire_idx(b + 2, par)

        qsl = pl.ds(b * BLK_Q, BLK_Q)
        qx = q_v[0, qsl]
        qy = q_v[1, qsl]
        qz = q_v[2, qsl]
        h = q_v[3, qsl]
        neg_inv_h2 = -1.0 / (h * h + EPS_CONST)
        two_inv = 2.0 * neg_inv_h2

        zf = jnp.zeros((16,), jnp.float32)
        acc0 = (zf,) * 11

        def base_k(k, acc):
            (sw, swf, sgx, sgy, sgz, sfx, sfy, sfz, swx, swy, swz) = acc
            row = lane64 + k
            nbx = plsc.load_gather(rows_b, [row, cols[0]])
            nby = plsc.load_gather(rows_b, [row, cols[1]])
            nbz = plsc.load_gather(rows_b, [row, cols[2]])
            nnx = plsc.load_gather(rows_b, [row, cols[3]])
            nny = plsc.load_gather(rows_b, [row, cols[4]])
            nnz = plsc.load_gather(rows_b, [row, cols[5]])
            dx = qx - nbx
            dy = qy - nby
            dz = qz - nbz
            d2 = dx * dx + dy * dy + dz * dz
            bw = jnp.exp(d2 * neg_inv_h2)
            f = dx * nnx + dy * nny + dz * nnz
            s = bw * two_inv
            bgx = s * dx
            bgy = s * dy
            bgz = s * dz
            der_v[0, k, :] = bw
            der_v[1, k, :] = f
            der_v[2, k, :] = bgx
            der_v[3, k, :] = bgy
            der_v[4, k, :] = bgz
            der_v[5, k, :] = nnx
            der_v[6, k, :] = nny
            der_v[7, k, :] = nnz
            return (sw + bw, swf + bw * f,
                    sgx + bgx, sgy + bgy, sgz + bgz,
                    sfx + bgx * f, sfy + bgy * f, sfz + bgz * f,
                    swx + bw * nnx, swy + bw * nny, swz + bw * nnz)

        acc = plsc.parallel_loop(0, K_NB, 1, unroll=UNROLL, carry=acc0)(base_k)

        def finish(acc):
            (sw, swf, sgx, sgy, sgz, sfx, sfy, sfz, swx, swy, swz) = acc
            swe = sw + EPS_CONST
            inv = 1.0 / swe
            pot = swf * inv
            gx = (sfx + swx - sgx * pot) * inv
            gy = (sfy + swy - sgy * pot) * inv
            gz = (sfz + swz - sgz * pot) * inv
            return pot, gx, gy, gz

        pot, gx, gy, gz = finish(acc)

        for _ in range(2):
            def iter_k(k, acc, gx=gx, gy=gy, gz=gz):
                (sw, swf, sgx, sgy, sgz,
                 sfx, sfy, sfz, swx, swy, swz) = acc
                bw = der_v[0, k, :]
                f = der_v[1, k, :]
                bgx = der_v[2, k, :]
                bgy = der_v[3, k, :]
                bgz = der_v[4, k, :]
                nnx = der_v[5, k, :]
                nny = der_v[6, k, :]
                nnz = der_v[7, k, :]
                tx = nnx - gx
                ty = nny - gy
                tz = nnz - gz
                nd2 = tx * tx + ty * ty + tz * tz
                rw = jnp.exp(nd2 * neg_inv_sig)
                w = bw * rw
                wgx = bgx * rw
                wgy = bgy * rw
                wgz = bgz * rw
                return (sw + w, swf + w * f,
                        sgx + wgx, sgy + wgy, sgz + wgz,
                        sfx + wgx * f, sfy + wgy * f, sfz + wgz * f,
                        swx + w * nnx, swy + w * nny, swz + w * nnz)

            acc = plsc.parallel_loop(0, K_NB, 1, unroll=UNROLL, carry=acc0)(iter_k)
            pot, gx, gy, gz = finish(acc)

        out_v[0, qsl] = pot
        out_v[1, qsl] = gx
        out_v[2, qsl] = gy
        out_v[3, qsl] = gz
        return _

    lax.fori_loop(0, NBLK, block_body, 0)

    for i, dst_h in enumerate((pot_hbm, gx_hbm, gy_hbm, gz_hbm)):
        pltpu.sync_copy(out_v.at[i], dst_h.at[pl.ds(qbase, QPT)])


@jax.jit
def _run(packed, qx, qy, qz, h, idx2):
    mesh = plsc.VectorSubcoreMesh(core_axis_name="c", subcore_axis_name="s")
    kfn = pl.kernel(
        _sc_body,
        out_type=tuple(
            jax.ShapeDtypeStruct((Q_TOTAL,), jnp.float32) for _ in range(4)
        ),
        mesh=mesh,
        compiler_params=pltpu.CompilerParams(
            needs_layout_passes=False, use_tc_tiling_on_sc=False),
        scratch_types=(
            pltpu.VMEM((2, NCHUNK, IDX_CHUNK), jnp.int32),
            pltpu.VMEM((2, ROWS_PER_BLK, 8), jnp.float32),
            pltpu.VMEM((8, K_NB, 16), jnp.float32),
            pltpu.VMEM((4, QPT), jnp.float32),
            pltpu.VMEM((4, QPT), jnp.float32),
            pltpu.SemaphoreType.DMA,
            pltpu.SemaphoreType.DMA,
        ),
    )
    return kfn(packed, qx, qy, qz, h, idx2)


def kernel(query_points, source_vertices, source_normals, neighbor_indices,
           bandwidth_h, compute_gradient):
    n = source_vertices.shape[0]
    packed = jnp.concatenate(
        [source_vertices, source_normals,
         jnp.zeros((n, 2), jnp.float32)], axis=1)
    idx2 = neighbor_indices.astype(jnp.int32).reshape(-1, IDX_CHUNK)
    pot, gx, gy, gz = _run(
        packed, query_points[:, 0], query_points[:, 1], query_points[:, 2],
        bandwidth_h, idx2)
    grad = jnp.stack([gx, gy, gz], axis=1)
    grad = jnp.where(compute_gradient != 0, grad, jnp.zeros_like(grad))
    return (pot, grad)
